# baseline (speedup 1.0000x reference)
import jax
import jax.numpy as jnp
from jax.experimental import pallas as pl
from jax.experimental.pallas import tpu as pltpu

_HID = 32
_LN_EPS = 1e-5
_NSTRIP = 6


def _ln_core(e):
    mu = jnp.mean(e, axis=0, keepdims=True)
    d = e - mu
    var = jnp.mean(d * d, axis=0, keepdims=True)
    return d * jax.lax.rsqrt(var + _LN_EPS)


def _dotb(a, b):
    return jnp.dot(a.astype(jnp.bfloat16), b.astype(jnp.bfloat16),
                   preferred_element_type=jnp.float32)


def _silu(h):
    return 0.5 * h * (jnp.tanh(0.5 * h) + 1.0)


def _emb_core(x, w1, b1, w2, b2):
    h = _dotb(w1, x) + b1
    h = _silu(h)
    e = _dotb(w2, h) + b2
    return _ln_core(e)


def _body(xg_hbm, xm_hbm, w1g_ref, w2g_ref, w1m_ref, w2m_ref, w1e_ref,
          w2e_ref, wa_ref, wb_ref, wc_ref, vg_ref, vgt_ref, vm_ref, vmt_ref,
          out_ref, xg_ref, xm_ref, s_ref, sem_t, sem_in):
    n_grid = xg_hbm.shape[1]
    n_mesh = xm_hbm.shape[1]
    n_out = out_ref.shape[1]
    a_end = (n_grid // 128) * 128
    b_end = ((n_grid + n_mesh) // 128) * 128
    w = (n_grid // _NSTRIP // 128 + 1) * 128
    bounds = [min(s * w, n_grid) for s in range(_NSTRIP + 1)]

    in_cps = []
    for s in range(_NSTRIP):
        lo, hi = bounds[s], bounds[s + 1]
        cp = pltpu.make_async_copy(xg_hbm.at[:, pl.ds(lo, hi - lo)],
                                   xg_ref.at[:, pl.ds(lo, hi - lo)],
                                   sem_in.at[s])
        cp.start()
        in_cps.append(cp)
    cp_xm = pltpu.make_async_copy(xm_hbm, xm_ref, sem_in.at[_NSTRIP])
    cp_xm.start()

    vg = vg_ref[...]
    vgt = vgt_ref[...]
    wc = wc_ref[...]
    dot = lambda a, b: jnp.dot(a, b, preferred_element_type=jnp.float32)

    wfold = dot(wc, dot(wb_ref[...], wa_ref[...]))
    g_g, b_g = vgt[2:3, :], vg[:, 3:4]
    g_e, b_e = vgt[6:7, :], vg[:, 7:8]
    wfold_g = wfold * g_g
    c_top = dot(wfold, b_g)
    w1e_g = w1e_ref[...] * g_g
    c1e = dot(w1e_ref[...], b_g) + vg[:, 4:5]
    wc_ge = wc * g_e
    c_bot = dot(wc, b_e)

    cps = []

    def flush(lo, hi):
        cp = pltpu.make_async_copy(s_ref.at[:, pl.ds(lo, hi - lo)],
                                   out_ref.at[:, pl.ds(lo, hi - lo)],
                                   sem_t.at[len(cps)])
        cp.start()
        cps.append(cp)

    zs = []
    for s in range(_NSTRIP):
        lo, hi = bounds[s], bounds[s + 1]
        in_cps[s].wait()
        z = _emb_core(xg_ref[:, lo:hi], w1g_ref[...], vg[:, 0:1],
                      w2g_ref[...], vg[:, 1:2])
        zs.append(z)
        s_ref[:, lo:hi] = _dotb(wfold_g, z) + c_top
        flush(lo, min(hi, a_end))

    vm = vm_ref[...]
    vmt = vmt_ref[...]
    g_m, b_m = vmt[2:3, :], vm[:, 3:4]
    cp_xm.wait()
    z_m = _emb_core(xm_ref[...], w1m_ref[...], vm[:, 0:1], w2m_ref[...],
                    vm[:, 1:2])
    s_ref[:, n_grid:n_grid + n_mesh] = _dotb(wfold * g_m, z_m) + dot(wfold, b_m)
    flush(a_end, b_end)

    base = n_grid + n_mesh
    for s in range(_NSTRIP):
        lo, hi = bounds[s], bounds[s + 1]
        h = _dotb(w1e_g, zs[s]) + c1e
        h = _silu(h)
        ze = _ln_core(_dotb(w2e_ref[...], h) + vg[:, 5:6])
        s_ref[:, base + lo:base + hi] = _dotb(wc_ge, ze) + c_bot
        c_lo = b_end if s == 0 else b_end + bounds[s]
        c_hi = n_out if s == _NSTRIP - 1 else b_end + bounds[s + 1]
        flush(c_lo, c_hi)

    for cp in cps:
        cp.wait()


def kernel(g2m_features, g2m_edge_index, grid_features, m2g_features,
           m2g_edge_index, m2m_features, mesh_static_features, m2m_edge_index,
           params):
    n_grid, grid_dim = grid_features.shape
    n_mesh, mesh_dim = mesh_static_features.shape
    n_out = n_grid + n_mesh + n_grid

    pg = params["grid_embedder"]
    pm = params["mesh_embedder"]
    pe = params["encoding_grid_mlp"]
    wa = params["g2m_gnn"]["rx_node"]["W"]
    wb = params["processor"]["rx_node"]["W"]
    wc = params["m2g_gnn"]["rx_node"]["W"]

    gvecs = [
        pg["layers"][0]["b"], pg["layers"][1]["b"], pg["ln"]["g"], pg["ln"]["b"],
        pe["layers"][0]["b"], pe["layers"][1]["b"], pe["ln"]["g"], pe["ln"]["b"],
    ]
    mvecs = [
        pm["layers"][0]["b"], pm["layers"][1]["b"], pm["ln"]["g"], pm["ln"]["b"],
    ]
    vg = jnp.stack(gvecs, axis=1)
    vgt = jnp.stack(gvecs, axis=0)
    vm = jnp.stack(mvecs, axis=1)
    vmt = jnp.stack(mvecs, axis=0)

    out_t = pl.pallas_call(
        _body,
        in_specs=([pl.BlockSpec(memory_space=pl.ANY)] * 2
                  + [pl.BlockSpec()] * 13),
        out_specs=pl.BlockSpec(memory_space=pl.ANY),
        out_shape=jax.ShapeDtypeStruct((_HID, n_out), jnp.float32),
        scratch_shapes=[
            pltpu.VMEM((grid_dim, n_grid), jnp.float32),
            pltpu.VMEM((mesh_dim, n_mesh), jnp.float32),
            pltpu.VMEM((_HID, n_out), jnp.float32),
            pltpu.SemaphoreType.DMA((2 * _NSTRIP + 1,)),
            pltpu.SemaphoreType.DMA((_NSTRIP + 1,)),
        ],
    )(grid_features.T, mesh_static_features.T,
      pg["layers"][0]["W"], pg["layers"][1]["W"],
      pm["layers"][0]["W"], pm["layers"][1]["W"],
      pe["layers"][0]["W"], pe["layers"][1]["W"],
      wa, wb, wc, vg, vgt, vm, vmt)

    return out_t.T


# speedup vs baseline: 1.0588x; 1.0588x over previous
import jax
import jax.numpy as jnp
from jax.experimental import pallas as pl
from jax.experimental.pallas import tpu as pltpu

_HID = 32
_LN_EPS = 1e-5
_NSTRIP = 6


def _ln_core(e):
    mu = jnp.mean(e, axis=0, keepdims=True)
    d = e - mu
    var = jnp.mean(d * d, axis=0, keepdims=True)
    return d * jax.lax.rsqrt(var + _LN_EPS)


def _dotb(a, b):
    return jnp.dot(a.astype(jnp.bfloat16), b.astype(jnp.bfloat16),
                   preferred_element_type=jnp.float32)


def _silu(h):
    return 0.5 * h * (jnp.tanh(0.5 * h) + 1.0)


def _emb_core(x, w1, b1, w2, b2):
    h = _dotb(w1, x) + b1
    h = _silu(h)
    e = _dotb(w2, h) + b2
    return _ln_core(e)


def _body(xg_ref, xm_ref, w1g_ref, w2g_ref, w1m_ref, w2m_ref, w1e_ref,
          w2e_ref, wa_ref, wb_ref, wc_ref, vg_ref, vgt_ref, vm_ref, vmt_ref,
          out_ref, s_ref, sem_t):
    n_grid = xg_ref.shape[1]
    n_mesh = xm_ref.shape[1]
    n_out = out_ref.shape[1]
    a_end = (n_grid // 128) * 128
    b_end = ((n_grid + n_mesh) // 128) * 128
    w = (n_grid // _NSTRIP // 128 + 1) * 128
    bounds = [min(s * w, n_grid) for s in range(_NSTRIP + 1)]

    vg = vg_ref[...]
    vgt = vgt_ref[...]
    wc = wc_ref[...]
    dot = lambda a, b: jnp.dot(a, b, preferred_element_type=jnp.float32)

    wfold = dot(wc, dot(wb_ref[...], wa_ref[...]))
    g_g, b_g = vgt[2:3, :], vg[:, 3:4]
    g_e, b_e = vgt[6:7, :], vg[:, 7:8]
    wfold_g = wfold * g_g
    c_top = dot(wfold, b_g)
    w1e_g = w1e_ref[...] * g_g
    c1e = dot(w1e_ref[...], b_g) + vg[:, 4:5]
    wc_ge = wc * g_e
    c_bot = dot(wc, b_e)

    cps = []

    def flush(lo, hi):
        cp = pltpu.make_async_copy(s_ref.at[:, pl.ds(lo, hi - lo)],
                                   out_ref.at[:, pl.ds(lo, hi - lo)],
                                   sem_t.at[len(cps)])
        cp.start()
        cps.append(cp)

    zs = []
    for s in range(_NSTRIP):
        lo, hi = bounds[s], bounds[s + 1]
        z = _emb_core(xg_ref[:, lo:hi], w1g_ref[...], vg[:, 0:1],
                      w2g_ref[...], vg[:, 1:2])
        zs.append(z)
        s_ref[:, lo:hi] = _dotb(wfold_g, z) + c_top
        flush(lo, min(hi, a_end))

    vm = vm_ref[...]
    vmt = vmt_ref[...]
    g_m, b_m = vmt[2:3, :], vm[:, 3:4]
    z_m = _emb_core(xm_ref[...], w1m_ref[...], vm[:, 0:1], w2m_ref[...],
                    vm[:, 1:2])
    s_ref[:, n_grid:n_grid + n_mesh] = _dotb(wfold * g_m, z_m) + dot(wfold, b_m)
    flush(a_end, b_end)

    base = n_grid + n_mesh
    for s in range(_NSTRIP):
        lo, hi = bounds[s], bounds[s + 1]
        h = _dotb(w1e_g, zs[s]) + c1e
        h = _silu(h)
        ze = _ln_core(_dotb(w2e_ref[...], h) + vg[:, 5:6])
        s_ref[:, base + lo:base + hi] = _dotb(wc_ge, ze) + c_bot
        c_lo = b_end if s == 0 else b_end + bounds[s]
        c_hi = n_out if s == _NSTRIP - 1 else b_end + bounds[s + 1]
        flush(c_lo, c_hi)

    for cp in cps:
        cp.wait()


def kernel(g2m_features, g2m_edge_index, grid_features, m2g_features,
           m2g_edge_index, m2m_features, mesh_static_features, m2m_edge_index,
           params):
    n_grid, grid_dim = grid_features.shape
    n_mesh, mesh_dim = mesh_static_features.shape
    n_out = n_grid + n_mesh + n_grid

    pg = params["grid_embedder"]
    pm = params["mesh_embedder"]
    pe = params["encoding_grid_mlp"]
    wa = params["g2m_gnn"]["rx_node"]["W"]
    wb = params["processor"]["rx_node"]["W"]
    wc = params["m2g_gnn"]["rx_node"]["W"]

    gvecs = [
        pg["layers"][0]["b"], pg["layers"][1]["b"], pg["ln"]["g"], pg["ln"]["b"],
        pe["layers"][0]["b"], pe["layers"][1]["b"], pe["ln"]["g"], pe["ln"]["b"],
    ]
    mvecs = [
        pm["layers"][0]["b"], pm["layers"][1]["b"], pm["ln"]["g"], pm["ln"]["b"],
    ]
    vg = jnp.stack(gvecs, axis=1)
    vgt = jnp.stack(gvecs, axis=0)
    vm = jnp.stack(mvecs, axis=1)
    vmt = jnp.stack(mvecs, axis=0)

    out_t = pl.pallas_call(
        _body,
        out_specs=pl.BlockSpec(memory_space=pl.ANY),
        out_shape=jax.ShapeDtypeStruct((_HID, n_out), jnp.float32),
        scratch_shapes=[
            pltpu.VMEM((_HID, n_out), jnp.float32),
            pltpu.SemaphoreType.DMA((2 * _NSTRIP + 1,)),
        ],
    )(grid_features.T, mesh_static_features.T,
      pg["layers"][0]["W"], pg["layers"][1]["W"],
      pm["layers"][0]["W"], pm["layers"][1]["W"],
      pe["layers"][0]["W"], pe["layers"][1]["W"],
      wa, wb, wc, vg, vgt, vm, vmt)

    return out_t.T
